# Initial kernel scaffold; baseline (speedup 1.0000x reference)
import jax
import jax.numpy as jnp
from jax.experimental import pallas as pl


def kernel(positions, hidden_states, residual, qkv_w, out_w, q_ln, k_ln, op_w, ffn_w, gate_w, e_bias, w1, w2):
    raise NotImplementedError("write your pallas kernel here")


# speedup vs baseline: 2.8604x; 2.8604x over previous
import functools
import math

import jax
import jax.numpy as jnp
from jax import lax
from jax.experimental import pallas as pl
from jax.experimental.pallas import tpu as pltpu

T = 2048
HID = 1024
NH = 16
NKV = 4
HD = 64
E = 64
K = 2
FF = 512
EPS = 1e-05
THETA = 1000000.0
QKV = (NH + 2 * NKV) * HD

TB = 256
TM = 128
G = 96
F32 = jnp.float32


def _rms(x, w, eps=EPS):
    return x * lax.rsqrt(jnp.mean(x * x, axis=-1, keepdims=True) + eps) * w


def _dot_t(a, b):
    return lax.dot_general(a, b, (((1,), (1,)), ((), ())),
                           preferred_element_type=F32)


def _prologue_body(pos_ref, hs_ref, rs_ref, qkvw_ref, opw_ref, qln_ref,
                   kln_ref, res_ref, q_ref, k_ref, v_ref):
    res = hs_ref[...] + rs_ref[...]
    res_ref[...] = res
    h = _rms(res, opw_ref[...])
    qkv = _dot_t(h, qkvw_ref[...])

    posf = pos_ref[0, 0, :].astype(F32)
    i2 = lax.broadcasted_iota(jnp.int32, (TB, HD // 2), 1).astype(F32)
    inv = jnp.exp(i2 * (-2.0 * math.log(THETA) / HD))
    f = posf[:, None] * inv
    cosf = jnp.concatenate([jnp.cos(f), jnp.cos(f)], axis=1)
    sinf = jnp.concatenate([jnp.sin(f), jnp.sin(f)], axis=1)

    def rope(x):
        x1 = x[:, :HD // 2]
        x2 = x[:, HD // 2:]
        return x * cosf + jnp.concatenate([-x2, x1], axis=1) * sinf

    qs = []
    for hh in range(NH):
        qh = qkv[:, hh * HD:(hh + 1) * HD]
        qs.append(rope(_rms(qh, qln_ref[...])))
    q_ref[...] = jnp.concatenate(qs, axis=1)

    ks = []
    for hh in range(NKV):
        kh = qkv[:, NH * HD + hh * HD:NH * HD + (hh + 1) * HD]
        ks.append(rope(_rms(kh, kln_ref[...])))
    k_ref[...] = jnp.concatenate(ks, axis=1)

    v_ref[...] = qkv[:, (NH + NKV) * HD:]


def _attn_body(q_ref, k_ref, v_ref, o_ref):
    qi = pl.program_id(0)
    scale = HD ** -0.5
    rows = lax.broadcasted_iota(jnp.int32, (TB, T), 0) + qi * TB
    cols = lax.broadcasted_iota(jnp.int32, (TB, T), 1)
    mask = rows >= cols
    outs = []
    for hh in range(NH):
        qh = q_ref[:, hh * HD:(hh + 1) * HD]
        kv = hh // (NH // NKV)
        kh = k_ref[:, kv * HD:(kv + 1) * HD]
        vh = v_ref[:, kv * HD:(kv + 1) * HD]
        s = _dot_t(qh, kh) * scale
        s = jnp.where(mask, s, -1e30)
        m = jnp.max(s, axis=1, keepdims=True)
        e = jnp.exp(s - m)
        p = e / jnp.sum(e, axis=1, keepdims=True)
        outs.append(lax.dot_general(p, vh, (((1,), (0,)), ((), ())),
                                    preferred_element_type=F32))
    o_ref[...] = jnp.concatenate(outs, axis=1)


def _epilogue_body(o_ref, ow_ref, res_ref, ffnw_ref, gw_ref, eb_ref,
                   res2_ref, h2_ref, ids_ref, tw_ref):
    res2 = _dot_t(o_ref[...], ow_ref[...]) + res_ref[...]
    res2_ref[...] = res2
    h2 = _rms(res2, ffnw_ref[...])
    h2_ref[...] = h2
    logits = _dot_t(h2, gw_ref[...])
    scores = jax.nn.sigmoid(logits)
    sb = scores + eb_ref[...]
    iota = lax.broadcasted_iota(jnp.int32, (TB, E), 1)

    m1 = jnp.max(sb, axis=1, keepdims=True)
    is1 = sb == m1
    i1 = jnp.min(jnp.where(is1, iota, E), axis=1)
    sel1 = iota == i1[:, None]
    t1 = jnp.sum(jnp.where(sel1, scores, 0.0), axis=1)

    sb2 = jnp.where(sel1, -1e30, sb)
    m2 = jnp.max(sb2, axis=1, keepdims=True)
    is2 = sb2 == m2
    i2 = jnp.min(jnp.where(is2, iota, E), axis=1)
    sel2 = iota == i2[:, None]
    t2 = jnp.sum(jnp.where(sel2, scores, 0.0), axis=1)

    den = t1 + t2
    ids_ref[...] = jnp.stack([i1, i2], axis=1)
    tw_ref[...] = jnp.stack([t1 / den, t2 / den], axis=1)


def _moe_body(gt_ref, tok_ref, nt_ref, h2_ref, w1_ref, w2_ref, ws_ref,
              out_ref, x_ref):
    t = pl.program_id(0)

    @pl.when(t == 0)
    def _():
        out_ref[...] = jnp.zeros((T, HID), F32)

    @pl.when(t < nt_ref[0])
    def _():
        def gath(r, c):
            idx = tok_ref[t * TM + r]
            x_ref[pl.ds(r, 1), :] = h2_ref[pl.ds(idx, 1), :]
            return c
        lax.fori_loop(0, TM, gath, 0)

        x = x_ref[...]
        g = _dot_t(x, w1_ref[0])
        a = jax.nn.silu(g[:, :FF]) * g[:, FF:]
        y = _dot_t(a, w2_ref[0])
        x_ref[...] = y * ws_ref[0, 0][:, None]

        def scat(r, c):
            idx = tok_ref[t * TM + r]
            row = x_ref[pl.ds(r, 1), :]
            out_ref[pl.ds(idx, 1), :] = out_ref[pl.ds(idx, 1), :] + row
            return c
        lax.fori_loop(0, TM, scat, 0)


def kernel(positions, hidden_states, residual, qkv_w, out_w, q_ln, k_ln,
           op_w, ffn_w, gate_w, e_bias, w1, w2):
    pos3 = positions.reshape(T // TB, 1, TB)
    opw2 = op_w.reshape(1, HID)
    qln2 = q_ln.reshape(1, HD)
    kln2 = k_ln.reshape(1, HD)
    ffnw2 = ffn_w.reshape(1, HID)
    eb2 = e_bias.reshape(1, E)

    res, q, k, v = pl.pallas_call(
        _prologue_body,
        grid=(T // TB,),
        in_specs=[
            pl.BlockSpec((1, 1, TB), lambda i: (i, 0, 0)),
            pl.BlockSpec((TB, HID), lambda i: (i, 0)),
            pl.BlockSpec((TB, HID), lambda i: (i, 0)),
            pl.BlockSpec((QKV, HID), lambda i: (0, 0)),
            pl.BlockSpec((1, HID), lambda i: (0, 0)),
            pl.BlockSpec((1, HD), lambda i: (0, 0)),
            pl.BlockSpec((1, HD), lambda i: (0, 0)),
        ],
        out_specs=[
            pl.BlockSpec((TB, HID), lambda i: (i, 0)),
            pl.BlockSpec((TB, NH * HD), lambda i: (i, 0)),
            pl.BlockSpec((TB, NKV * HD), lambda i: (i, 0)),
            pl.BlockSpec((TB, NKV * HD), lambda i: (i, 0)),
        ],
        out_shape=[
            jax.ShapeDtypeStruct((T, HID), F32),
            jax.ShapeDtypeStruct((T, NH * HD), F32),
            jax.ShapeDtypeStruct((T, NKV * HD), F32),
            jax.ShapeDtypeStruct((T, NKV * HD), F32),
        ],
    )(pos3, hidden_states, residual, qkv_w, opw2, qln2, kln2)

    o = pl.pallas_call(
        _attn_body,
        grid=(T // TB,),
        in_specs=[
            pl.BlockSpec((TB, NH * HD), lambda i: (i, 0)),
            pl.BlockSpec((T, NKV * HD), lambda i: (0, 0)),
            pl.BlockSpec((T, NKV * HD), lambda i: (0, 0)),
        ],
        out_specs=pl.BlockSpec((TB, NH * HD), lambda i: (i, 0)),
        out_shape=jax.ShapeDtypeStruct((T, NH * HD), F32),
    )(q, k, v)

    res2, h2, ids, tw = pl.pallas_call(
        _epilogue_body,
        grid=(T // TB,),
        in_specs=[
            pl.BlockSpec((TB, NH * HD), lambda i: (i, 0)),
            pl.BlockSpec((HID, NH * HD), lambda i: (0, 0)),
            pl.BlockSpec((TB, HID), lambda i: (i, 0)),
            pl.BlockSpec((1, HID), lambda i: (0, 0)),
            pl.BlockSpec((E, HID), lambda i: (0, 0)),
            pl.BlockSpec((1, E), lambda i: (0, 0)),
        ],
        out_specs=[
            pl.BlockSpec((TB, HID), lambda i: (i, 0)),
            pl.BlockSpec((TB, HID), lambda i: (i, 0)),
            pl.BlockSpec((TB, K), lambda i: (i, 0)),
            pl.BlockSpec((TB, K), lambda i: (i, 0)),
        ],
        out_shape=[
            jax.ShapeDtypeStruct((T, HID), F32),
            jax.ShapeDtypeStruct((T, HID), F32),
            jax.ShapeDtypeStruct((T, K), jnp.int32),
            jax.ShapeDtypeStruct((T, K), F32),
        ],
    )(o, out_w, res, ffnw2, gate_w, eb2)

    flat_e = ids.reshape(-1)
    flat_w = tw.reshape(-1)
    order = jnp.argsort(flat_e, stable=True)
    sorted_tok = (order // K).astype(jnp.int32)
    sorted_w = flat_w[order]

    counts = jnp.zeros((E,), jnp.int32).at[flat_e].add(1)
    starts = jnp.concatenate([jnp.zeros((1,), jnp.int32),
                              jnp.cumsum(counts)[:-1]])
    tiles_pg = (counts + TM - 1) // TM
    first_tile = jnp.concatenate([jnp.zeros((1,), jnp.int32),
                                  jnp.cumsum(tiles_pg)[:-1]])
    n_tiles = jnp.sum(tiles_pg).astype(jnp.int32)

    tix = jnp.arange(G, dtype=jnp.int32)
    g_t = jnp.clip(jnp.searchsorted(first_tile, tix, side='right') - 1,
                   0, E - 1).astype(jnp.int32)
    within = tix - first_tile[g_t]
    row0 = starts[g_t] + within * TM
    slot = row0[:, None] + jnp.arange(TM, dtype=jnp.int32)[None, :]
    valid = (slot < (starts[g_t] + counts[g_t])[:, None]) \
        & (tix < n_tiles)[:, None]
    slot_c = jnp.clip(slot, 0, T * K - 1)
    tok_slot = jnp.where(valid, sorted_tok[slot_c], 0).reshape(-1)
    w_slot = jnp.where(valid, sorted_w[slot_c], 0.0).reshape(G, 1, TM)

    moe_out = pl.pallas_call(
        _moe_body,
        grid_spec=pltpu.PrefetchScalarGridSpec(
            num_scalar_prefetch=3,
            grid=(G,),
            in_specs=[
                pl.BlockSpec((T, HID), lambda t, gt, tok, nt: (0, 0)),
                pl.BlockSpec((1, 2 * FF, HID),
                             lambda t, gt, tok, nt: (gt[t], 0, 0)),
                pl.BlockSpec((1, HID, FF),
                             lambda t, gt, tok, nt: (gt[t], 0, 0)),
                pl.BlockSpec((1, 1, TM), lambda t, gt, tok, nt: (t, 0, 0)),
            ],
            out_specs=pl.BlockSpec((T, HID), lambda t, gt, tok, nt: (0, 0)),
            scratch_shapes=[pltpu.VMEM((TM, HID), F32)],
        ),
        out_shape=jax.ShapeDtypeStruct((T, HID), F32),
    )(g_t, tok_slot, jnp.array([0], jnp.int32) + n_tiles,
      h2, w1, w2, w_slot)

    return (moe_out, res2)
